# baseline (speedup 1.0000x reference)
import functools

import jax
import jax.numpy as jnp
from jax import lax
from jax.experimental import pallas as pl
from jax.experimental.pallas import tpu as pltpu
from jax.experimental.pallas import tpu_sc as plsc

BATCH = 16384
FIELDS = 26
DIM = 64
NUM_IDS = BATCH * FIELDS

NC = 2
NS = 16
NW = NC * NS
BPW = NUM_IDS // NW
CHUNK = 128
NCH = BPW // CHUNK
NBUF = 4
NG = NCH // NBUF

assert BPW * NW == NUM_IDS and NCH * CHUNK == BPW and NG * NBUF == NCH

_mesh = plsc.VectorSubcoreMesh(
    core_axis_name="c", subcore_axis_name="s", num_cores=NC, num_subcores=NS
)


def _emb_body(table_hbm, idx_hbm, out_hbm, idx_v, rows_v, gsem, osem):
    cid = lax.axis_index("c")
    sid = lax.axis_index("s")
    wid = sid * NC + cid
    base = wid * BPW

    pltpu.sync_copy(idx_hbm.at[wid], idx_v)

    def gather_start(j, b):
        pltpu.async_copy(table_hbm.at[idx_v.at[j]], rows_v.at[b], gsem)

    def gather_wait(j, b):
        pltpu.make_async_copy(table_hbm.at[idx_v.at[j]], rows_v.at[b], gsem).wait()

    def out_start(j, b):
        pltpu.async_copy(rows_v.at[b], out_hbm.at[pl.ds(base + j * CHUNK, CHUNK)], osem)

    def out_wait(j, b):
        pltpu.make_async_copy(
            rows_v.at[b], out_hbm.at[pl.ds(base + j * CHUNK, CHUNK)], osem
        ).wait()

    for b in range(NBUF):
        gather_start(b, b)

    def body(g, carry):
        for b in range(NBUF):
            j = g * NBUF + b
            gather_wait(j, b)
            out_start(j, b)
            out_wait(j, b)
            gather_start(j + NBUF, b)
        return carry

    lax.fori_loop(0, NG - 1, body, 0)

    for b in range(NBUF):
        j = (NG - 1) * NBUF + b
        gather_wait(j, b)
        out_start(j, b)
    for b in range(NBUF):
        j = (NG - 1) * NBUF + b
        out_wait(j, b)


_SCRATCH = [
    pltpu.VMEM((NCH, CHUNK), jnp.int32),
    pltpu.VMEM((NBUF, CHUNK, DIM), jnp.float32),
    pltpu.SemaphoreType.DMA,
    pltpu.SemaphoreType.DMA,
]

_emb_lookup = pl.kernel(
    _emb_body,
    out_type=jax.ShapeDtypeStruct((NUM_IDS, DIM), jnp.float32),
    mesh=_mesh,
    scratch_types=_SCRATCH,
    compiler_params=pltpu.CompilerParams(use_tc_tiling_on_sc=False),
)


def kernel(x, kernel):
    idx = jnp.reshape(x.astype(jnp.int32), (NW, NCH, CHUNK))
    out = _emb_lookup(kernel, idx)
    return jnp.reshape(out, (BATCH, FIELDS, DIM))
